# baseline (speedup 1.0000x reference)
import functools

import jax
import jax.numpy as jnp
from jax import lax
from jax.experimental import pallas as pl
from jax.experimental.pallas import tpu as pltpu
from jax.experimental.pallas import tpu_sc as plsc

_N = 10000
_N_PAD = 10240
_NC = 2
_NS = 16
_NW = _NC * _NS
_CH = 128
_ROWS_PER_TILE = _N_PAD // _NS


def _make_sc_scatter(w: int, ch: int, nbuf: int, n_chunks: int, npass: int):
    mesh = plsc.VectorSubcoreMesh(core_axis_name="c", subcore_axis_name="s",
                                  num_cores=_NC, num_subcores=_NS)
    hc = n_chunks // npass
    assert n_chunks % npass == 0 and hc % nbuf == 0 and hc >= 2 * nbuf

    def body(y_hbm, src_hbm, dst_hbm, zeros_hbm, out_hbm,
             src_v, dst_v, *rest):
        bufs = rest[:nbuf]
        acc_sh = rest[nbuf]
        sems = rest[nbuf + 1:]
        c = lax.axis_index("c")
        s = lax.axis_index("s")
        gid = c * _NS + s
        row0 = s * _ROWS_PER_TILE

        pltpu.sync_copy(zeros_hbm.at[pl.ds(row0, _ROWS_PER_TILE), :],
                        acc_sh.at[pl.ds(row0, _ROWS_PER_TILE), :])

        for p in range(npass):
            if npass == 1:
                pltpu.sync_copy(src_hbm.at[gid], src_v)
                pltpu.sync_copy(dst_hbm.at[gid], dst_v)
            else:
                pltpu.sync_copy(src_hbm.at[gid, pl.ds(p * hc, hc), :], src_v)
                pltpu.sync_copy(dst_hbm.at[gid, pl.ds(p * hc, hc), :], dst_v)
            if p == 0:
                plsc.subcore_barrier()

            if nbuf == 1:
                @pl.loop(0, hc)
                def _chunk(j):
                    pltpu.async_copy(y_hbm.at[src_v.at[j]], bufs[0],
                                     sems[0]).wait()
                    pltpu.sync_copy(bufs[0], acc_sh.at[dst_v.at[j]], add=True)
            else:
                for b in range(nbuf):
                    pltpu.async_copy(y_hbm.at[src_v.at[b]], bufs[b], sems[b])

                @pl.loop(0, hc - nbuf, step=nbuf)
                def _chunk(j):
                    for b in range(nbuf):
                        jj = j + b
                        pltpu.make_async_copy(y_hbm.at[src_v.at[jj]],
                                              bufs[b], sems[b]).wait()
                        pltpu.sync_copy(bufs[b], acc_sh.at[dst_v.at[jj]],
                                        add=True)
                        pltpu.async_copy(y_hbm.at[src_v.at[jj + nbuf]],
                                         bufs[b], sems[b])

                for b in range(nbuf):
                    jj = hc - nbuf + b
                    pltpu.make_async_copy(y_hbm.at[src_v.at[jj]],
                                          bufs[b], sems[b]).wait()
                    pltpu.sync_copy(bufs[b], acc_sh.at[dst_v.at[jj]], add=True)

        plsc.subcore_barrier()
        pltpu.sync_copy(acc_sh.at[pl.ds(row0, _ROWS_PER_TILE), :],
                        out_hbm.at[c, pl.ds(row0, _ROWS_PER_TILE), :])

    return pl.kernel(
        body,
        out_type=jax.ShapeDtypeStruct((_NC, _N_PAD, w), jnp.float32),
        mesh=mesh,
        compiler_params=pltpu.CompilerParams(use_tc_tiling_on_sc=False),
        scratch_types=[
            pltpu.VMEM((hc, ch), jnp.int32),
            pltpu.VMEM((hc, ch), jnp.int32),
            *[pltpu.VMEM((ch, w), jnp.float32) for _ in range(nbuf)],
            pltpu.VMEM_SHARED((_N_PAD, w), jnp.float32),
            *[pltpu.SemaphoreType.DMA for _ in range(nbuf)],
        ],
    )


def _pre_body(h_ref, wr_ref, wo_ref, b_ref, yr_ref, yo_ref):
    h = h_ref[...]
    yr_ref[...] = jnp.dot(h, wr_ref[...], preferred_element_type=jnp.float32)
    yo_ref[...] = (jnp.dot(h, wo_ref[...], preferred_element_type=jnp.float32)
                   + b_ref[...])


def _bn_relu(z, g, be):
    mask = lax.broadcasted_iota(jnp.int32, (_N_PAD, 1), 0) < _N
    z = jnp.where(mask, z, 0.0)
    mu = jnp.sum(z, axis=0, keepdims=True) / _N
    d = jnp.where(mask, z - mu, 0.0)
    var = jnp.sum(d * d, axis=0, keepdims=True) / _N
    hn = (z - mu) * lax.rsqrt(var + 1e-5) * g + be
    hn = jnp.maximum(hn, 0.0)
    return jnp.where(mask, hn, 0.0)


def _mid_body(acc_ref, yo_ref, g_ref, be_ref, wr_ref, wo_ref, b_ref,
              yr2_ref, yo2_ref):
    z = acc_ref[0] + acc_ref[1] + yo_ref[...]
    h = _bn_relu(z, g_ref[...], be_ref[...])
    yr2_ref[...] = jnp.dot(h, wr_ref[...], preferred_element_type=jnp.float32)
    yo2_ref[...] = (jnp.dot(h, wo_ref[...], preferred_element_type=jnp.float32)
                    + b_ref[...])


def _post_body(acc_ref, yo_ref, g_ref, be_ref, h_ref):
    z = acc_ref[0] + acc_ref[1] + yo_ref[...]
    h_ref[...] = _bn_relu(z, g_ref[...], be_ref[...])


def _tc_call(body, out_shapes, *args):
    return pl.pallas_call(
        body,
        out_shape=out_shapes,
    )(*args)


def kernel(x, edge_index, params):
    n, d = x.shape
    e = edge_index.shape[1]
    ch1, nbuf1, npass1 = 128, 2, 2
    ch2, nbuf2, npass2 = 128, 2, 1
    quantum = _NW * 128
    n_chunks_tgt = 84
    e_pad = quantum * n_chunks_tgt
    assert e_pad >= e
    nck1 = e_pad // (_NW * ch1)
    nck2 = e_pad // (_NW * ch2)
    src = edge_index[0].astype(jnp.int32)
    dst = edge_index[1].astype(jnp.int32)
    pad_src = jnp.full((e_pad - e,), _N, jnp.int32)
    pad_dst = jnp.arange(e_pad - e, dtype=jnp.int32) % _N_PAD
    src = jnp.concatenate([src, pad_src])
    dst = jnp.concatenate([dst, pad_dst])
    src1 = src.reshape(_NW, nck1, ch1)
    dst1 = dst.reshape(_NW, nck1, ch1)
    src2 = src.reshape(_NW, nck2, ch2)
    dst2 = dst.reshape(_NW, nck2, ch2)

    xp = jnp.zeros((_N_PAD, d), jnp.float32).at[:n].set(x)

    scatter_128 = _make_sc_scatter(128, ch1, nbuf1, nck1, npass1)
    scatter_32 = _make_sc_scatter(32, ch2, nbuf2, nck2, npass2)
    zeros_128 = jnp.zeros((_N_PAD, 128), jnp.float32)
    zeros_32 = jnp.zeros((_N_PAD, 32), jnp.float32)

    xs = [xp]
    for p in params:
        h_cat = jnp.concatenate(xs, axis=-1)
        dmid = p['W1_rel'].shape[1]
        hdim = p['W2_rel'].shape[1]
        y1r, y1o = _tc_call(
            _pre_body,
            (jax.ShapeDtypeStruct((_N_PAD, dmid), jnp.float32),
             jax.ShapeDtypeStruct((_N_PAD, dmid), jnp.float32)),
            h_cat, p['W1_rel'], p['W1_root'], p['b1'].reshape(1, -1))
        acc1 = scatter_128(y1r, src1, dst1, zeros_128)
        y2r, y2o = _tc_call(
            _mid_body,
            (jax.ShapeDtypeStruct((_N_PAD, hdim), jnp.float32),
             jax.ShapeDtypeStruct((_N_PAD, hdim), jnp.float32)),
            acc1, y1o, p['g1'].reshape(1, -1), p['be1'].reshape(1, -1),
            p['W2_rel'], p['W2_root'], p['b2'].reshape(1, -1))
        acc2 = scatter_32(y2r, src2, dst2, zeros_32)
        h2 = _tc_call(
            _post_body,
            jax.ShapeDtypeStruct((_N_PAD, hdim), jnp.float32),
            acc2, y2o, p['g2'].reshape(1, -1), p['be2'].reshape(1, -1))
        xs.append(h2)

    return jnp.concatenate(xs, axis=-1)[:n]


# speedup vs baseline: 4.5531x; 4.5531x over previous
import functools

import jax
import jax.numpy as jnp
from jax import lax
from jax.experimental import pallas as pl
from jax.experimental.pallas import tpu as pltpu
from jax.experimental.pallas import tpu_sc as plsc

_N = 10000
_N_PAD = 10240
_NC = 2
_NS = 16
_NW = _NC * _NS
_CH = 128
_ROWS_PER_TILE = _N_PAD // _NS


def _make_sc_scatter(w: int, ch: int, nbuf: int, n_chunks: int, npass: int):
    mesh = plsc.VectorSubcoreMesh(core_axis_name="c", subcore_axis_name="s",
                                  num_cores=_NC, num_subcores=_NS)
    hc = n_chunks // npass
    assert n_chunks % npass == 0 and hc % nbuf == 0 and hc >= 2 * nbuf

    def body(y_hbm, src_hbm, dst_hbm, zeros_hbm, out_hbm,
             src_v, dst_v, *rest):
        bufs = rest[:nbuf]
        acc_sh = rest[nbuf]
        sems = rest[nbuf + 1:]
        c = lax.axis_index("c")
        s = lax.axis_index("s")
        gid = c * _NS + s
        row0 = s * _ROWS_PER_TILE

        pltpu.sync_copy(zeros_hbm.at[pl.ds(row0, _ROWS_PER_TILE), :],
                        acc_sh.at[pl.ds(row0, _ROWS_PER_TILE), :])

        for p in range(npass):
            if npass == 1:
                pltpu.sync_copy(src_hbm.at[gid], src_v)
                pltpu.sync_copy(dst_hbm.at[gid], dst_v)
            else:
                pltpu.sync_copy(src_hbm.at[gid, pl.ds(p * hc, hc), :], src_v)
                pltpu.sync_copy(dst_hbm.at[gid, pl.ds(p * hc, hc), :], dst_v)
            if p == 0:
                plsc.subcore_barrier()

            if nbuf == 1:
                @pl.loop(0, hc)
                def _chunk(j):
                    pltpu.async_copy(y_hbm.at[src_v.at[j]], bufs[0],
                                     sems[0]).wait()
                    pltpu.sync_copy(bufs[0], acc_sh.at[dst_v.at[j]], add=True)
            else:
                for b in range(nbuf):
                    pltpu.async_copy(y_hbm.at[src_v.at[b]], bufs[b], sems[b])

                @pl.loop(0, hc - nbuf, step=nbuf)
                def _chunk(j):
                    for b in range(nbuf):
                        jj = j + b
                        pltpu.make_async_copy(y_hbm.at[src_v.at[jj]],
                                              bufs[b], sems[b]).wait()
                        pltpu.sync_copy(bufs[b], acc_sh.at[dst_v.at[jj]],
                                        add=True)
                        pltpu.async_copy(y_hbm.at[src_v.at[jj + nbuf]],
                                         bufs[b], sems[b])

                for b in range(nbuf):
                    jj = hc - nbuf + b
                    pltpu.make_async_copy(y_hbm.at[src_v.at[jj]],
                                          bufs[b], sems[b]).wait()
                    pltpu.sync_copy(bufs[b], acc_sh.at[dst_v.at[jj]], add=True)

        plsc.subcore_barrier()
        pltpu.sync_copy(acc_sh.at[pl.ds(row0, _ROWS_PER_TILE), :],
                        out_hbm.at[c, pl.ds(row0, _ROWS_PER_TILE), :])

    return pl.kernel(
        body,
        out_type=jax.ShapeDtypeStruct((_NC, _N_PAD, w), jnp.float32),
        mesh=mesh,
        compiler_params=pltpu.CompilerParams(use_tc_tiling_on_sc=False),
        scratch_types=[
            pltpu.VMEM((hc, ch), jnp.int32),
            pltpu.VMEM((hc, ch), jnp.int32),
            *[pltpu.VMEM((ch, w), jnp.float32) for _ in range(nbuf)],
            pltpu.VMEM_SHARED((_N_PAD, w), jnp.float32),
            *[pltpu.SemaphoreType.DMA for _ in range(nbuf)],
        ],
    )


def _pre_body(h_ref, wr_ref, wo_ref, b_ref, yr_ref, yo_ref):
    h = h_ref[...]
    yr_ref[...] = jnp.dot(h, wr_ref[...], preferred_element_type=jnp.float32)
    yo_ref[...] = (jnp.dot(h, wo_ref[...], preferred_element_type=jnp.float32)
                   + b_ref[...])


def _bn_relu(z, g, be):
    mask = lax.broadcasted_iota(jnp.int32, (_N_PAD, 1), 0) < _N
    z = jnp.where(mask, z, 0.0)
    mu = jnp.sum(z, axis=0, keepdims=True) / _N
    d = jnp.where(mask, z - mu, 0.0)
    var = jnp.sum(d * d, axis=0, keepdims=True) / _N
    hn = (z - mu) * lax.rsqrt(var + 1e-5) * g + be
    hn = jnp.maximum(hn, 0.0)
    return jnp.where(mask, hn, 0.0)


def _mid_body(acc_ref, yo_ref, g_ref, be_ref, wr_ref, wo_ref, b_ref,
              yr2_ref, yo2_ref):
    z = acc_ref[0] + acc_ref[1] + yo_ref[...]
    h = _bn_relu(z, g_ref[...], be_ref[...])
    yr2_ref[...] = jnp.dot(h, wr_ref[...], preferred_element_type=jnp.float32)
    yo2_ref[...] = (jnp.dot(h, wo_ref[...], preferred_element_type=jnp.float32)
                    + b_ref[...])


def _post_body(acc_ref, yo_ref, g_ref, be_ref, h_ref):
    z = acc_ref[0] + acc_ref[1] + yo_ref[...]
    h_ref[...] = _bn_relu(z, g_ref[...], be_ref[...])


def _tc_call(body, out_shapes, *args):
    return pl.pallas_call(
        body,
        out_shape=out_shapes,
    )(*args)


def kernel(x, edge_index, params):
    n, d = x.shape
    e = edge_index.shape[1]
    ch1, nbuf1, npass1 = 128, 1, 1
    ch2, nbuf2, npass2 = 128, 1, 1
    quantum = _NW * 128
    e_pad = ((e + quantum - 1) // quantum) * quantum
    nck1 = e_pad // (_NW * ch1)
    nck2 = e_pad // (_NW * ch2)
    src = edge_index[0].astype(jnp.int32)
    dst = edge_index[1].astype(jnp.int32)
    pad_src = _N + jnp.arange(e_pad - e, dtype=jnp.int32) % (_N_PAD - _N)
    pad_dst = jnp.arange(e_pad - e, dtype=jnp.int32) % _N_PAD
    src = jnp.concatenate([src, pad_src])
    dst = jnp.concatenate([dst, pad_dst])
    src1 = src.reshape(_NW, nck1, ch1)
    dst1 = dst.reshape(_NW, nck1, ch1)
    src2 = src.reshape(_NW, nck2, ch2)
    dst2 = dst.reshape(_NW, nck2, ch2)

    xp = jnp.zeros((_N_PAD, d), jnp.float32).at[:n].set(x)

    scatter_128 = _make_sc_scatter(128, ch1, nbuf1, nck1, npass1)
    scatter_32 = _make_sc_scatter(32, ch2, nbuf2, nck2, npass2)
    zeros_128 = jnp.zeros((_N_PAD, 128), jnp.float32)
    zeros_32 = jnp.zeros((_N_PAD, 32), jnp.float32)

    xs = [xp]
    for p in params:
        h_cat = jnp.concatenate(xs, axis=-1)
        dmid = p['W1_rel'].shape[1]
        hdim = p['W2_rel'].shape[1]
        y1r, y1o = _tc_call(
            _pre_body,
            (jax.ShapeDtypeStruct((_N_PAD, dmid), jnp.float32),
             jax.ShapeDtypeStruct((_N_PAD, dmid), jnp.float32)),
            h_cat, p['W1_rel'], p['W1_root'], p['b1'].reshape(1, -1))
        acc1 = scatter_128(y1r, src1, dst1, zeros_128)
        y2r, y2o = _tc_call(
            _mid_body,
            (jax.ShapeDtypeStruct((_N_PAD, hdim), jnp.float32),
             jax.ShapeDtypeStruct((_N_PAD, hdim), jnp.float32)),
            acc1, y1o, p['g1'].reshape(1, -1), p['be1'].reshape(1, -1),
            p['W2_rel'], p['W2_root'], p['b2'].reshape(1, -1))
        acc2 = scatter_32(y2r, src2, dst2, zeros_32)
        h2 = _tc_call(
            _post_body,
            jax.ShapeDtypeStruct((_N_PAD, hdim), jnp.float32),
            acc2, y2o, p['g2'].reshape(1, -1), p['be2'].reshape(1, -1))
        xs.append(h2)

    return jnp.concatenate(xs, axis=-1)[:n]


# speedup vs baseline: 6.4307x; 1.4124x over previous
import functools

import jax
import jax.numpy as jnp
from jax import lax
from jax.experimental import pallas as pl
from jax.experimental.pallas import tpu as pltpu
from jax.experimental.pallas import tpu_sc as plsc

_N = 10000
_N_PAD = 10240
_NC = 2
_NS = 16
_NW = _NC * _NS
_CH = 128
_ROWS_PER_TILE = _N_PAD // _NS


def _make_sc_scatter(w: int, ch: int, nbuf: int, n_chunks: int, npass: int):
    mesh = plsc.VectorSubcoreMesh(core_axis_name="c", subcore_axis_name="s",
                                  num_cores=_NC, num_subcores=_NS)
    hc = n_chunks // npass
    assert n_chunks % npass == 0 and hc % nbuf == 0 and hc >= 2 * nbuf

    def body(y_hbm, src_hbm, dst_hbm, zeros_hbm, out_hbm,
             src_v, dst_v, *rest):
        bufs = rest[:nbuf]
        acc_sh = rest[nbuf]
        sems = rest[nbuf + 1:]
        c = lax.axis_index("c")
        s = lax.axis_index("s")
        gid = c * _NS + s
        row0 = s * _ROWS_PER_TILE

        pltpu.sync_copy(zeros_hbm.at[pl.ds(row0, _ROWS_PER_TILE), :],
                        acc_sh.at[pl.ds(row0, _ROWS_PER_TILE), :])

        for p in range(npass):
            if npass == 1:
                pltpu.sync_copy(src_hbm.at[gid], src_v)
                pltpu.sync_copy(dst_hbm.at[gid], dst_v)
            else:
                pltpu.sync_copy(src_hbm.at[gid, pl.ds(p * hc, hc), :], src_v)
                pltpu.sync_copy(dst_hbm.at[gid, pl.ds(p * hc, hc), :], dst_v)
            if p == 0:
                plsc.subcore_barrier()

            if nbuf == 1:
                @pl.loop(0, hc)
                def _chunk(j):
                    pltpu.async_copy(y_hbm.at[src_v.at[j]], bufs[0],
                                     sems[0]).wait()
                    pltpu.sync_copy(bufs[0], acc_sh.at[dst_v.at[j]], add=True)
            else:
                for b in range(nbuf):
                    pltpu.async_copy(y_hbm.at[src_v.at[b]], bufs[b], sems[b])

                @pl.loop(0, hc - nbuf, step=nbuf)
                def _chunk(j):
                    for b in range(nbuf):
                        jj = j + b
                        pltpu.make_async_copy(y_hbm.at[src_v.at[jj]],
                                              bufs[b], sems[b]).wait()
                        pltpu.sync_copy(bufs[b], acc_sh.at[dst_v.at[jj]],
                                        add=True)
                        pltpu.async_copy(y_hbm.at[src_v.at[jj + nbuf]],
                                         bufs[b], sems[b])

                for b in range(nbuf):
                    jj = hc - nbuf + b
                    pltpu.make_async_copy(y_hbm.at[src_v.at[jj]],
                                          bufs[b], sems[b]).wait()
                    pltpu.sync_copy(bufs[b], acc_sh.at[dst_v.at[jj]], add=True)

        plsc.subcore_barrier()
        pltpu.sync_copy(acc_sh.at[pl.ds(row0, _ROWS_PER_TILE), :],
                        out_hbm.at[c, pl.ds(row0, _ROWS_PER_TILE), :])

    return pl.kernel(
        body,
        out_type=jax.ShapeDtypeStruct((_NC, _N_PAD, w), jnp.float32),
        mesh=mesh,
        compiler_params=pltpu.CompilerParams(use_tc_tiling_on_sc=False),
        scratch_types=[
            pltpu.VMEM((hc, ch), jnp.int32),
            pltpu.VMEM((hc, ch), jnp.int32),
            *[pltpu.VMEM((ch, w), jnp.float32) for _ in range(nbuf)],
            pltpu.VMEM_SHARED((_N_PAD, w), jnp.float32),
            *[pltpu.SemaphoreType.DMA for _ in range(nbuf)],
        ],
    )


def _pre_body(h_ref, wr_ref, wo_ref, b_ref, yr_ref, yo_ref):
    h = h_ref[...]
    yr_ref[...] = jnp.dot(h, wr_ref[...], preferred_element_type=jnp.float32)
    yo_ref[...] = (jnp.dot(h, wo_ref[...], preferred_element_type=jnp.float32)
                   + b_ref[...])


def _bn_relu(z, g, be):
    mask = lax.broadcasted_iota(jnp.int32, (_N_PAD, 1), 0) < _N
    z = jnp.where(mask, z, 0.0)
    mu = jnp.sum(z, axis=0, keepdims=True) / _N
    d = jnp.where(mask, z - mu, 0.0)
    var = jnp.sum(d * d, axis=0, keepdims=True) / _N
    hn = (z - mu) * lax.rsqrt(var + 1e-5) * g + be
    hn = jnp.maximum(hn, 0.0)
    return jnp.where(mask, hn, 0.0)


def _mid_body(acc_ref, yo_ref, g_ref, be_ref, wr_ref, wo_ref, b_ref,
              yr2_ref, yo2_ref):
    z = acc_ref[0] + acc_ref[1] + yo_ref[...]
    h = _bn_relu(z, g_ref[...], be_ref[...])
    yr2_ref[...] = jnp.dot(h, wr_ref[...], preferred_element_type=jnp.float32)
    yo2_ref[...] = (jnp.dot(h, wo_ref[...], preferred_element_type=jnp.float32)
                    + b_ref[...])


def _post_body(acc_ref, yo_ref, g_ref, be_ref, h_ref):
    z = acc_ref[0] + acc_ref[1] + yo_ref[...]
    h_ref[...] = _bn_relu(z, g_ref[...], be_ref[...])


def _tc_call(body, out_shapes, *args):
    return pl.pallas_call(
        body,
        out_shape=out_shapes,
    )(*args)


def kernel(x, edge_index, params):
    n, d = x.shape
    e = edge_index.shape[1]
    ch1, nbuf1, npass1 = 128, 2, 2
    ch2, nbuf2, npass2 = 128, 2, 1
    quantum = _NW * 128 * 4
    e_pad = ((e + quantum - 1) // quantum) * quantum
    nck1 = e_pad // (_NW * ch1)
    nck2 = e_pad // (_NW * ch2)
    src = edge_index[0].astype(jnp.int32)
    dst = edge_index[1].astype(jnp.int32)
    pad_src = _N + jnp.arange(e_pad - e, dtype=jnp.int32) % (_N_PAD - _N)
    pad_dst = jnp.arange(e_pad - e, dtype=jnp.int32) % _N_PAD
    src = jnp.concatenate([src, pad_src])
    dst = jnp.concatenate([dst, pad_dst])
    src1 = src.reshape(_NW, nck1, ch1)
    dst1 = dst.reshape(_NW, nck1, ch1)
    src2 = src.reshape(_NW, nck2, ch2)
    dst2 = dst.reshape(_NW, nck2, ch2)

    xp = jnp.zeros((_N_PAD, d), jnp.float32).at[:n].set(x)

    scatter_128 = _make_sc_scatter(128, ch1, nbuf1, nck1, npass1)
    scatter_32 = _make_sc_scatter(32, ch2, nbuf2, nck2, npass2)
    zeros_128 = jnp.zeros((_N_PAD, 128), jnp.float32)
    zeros_32 = jnp.zeros((_N_PAD, 32), jnp.float32)

    xs = [xp]
    for p in params:
        h_cat = jnp.concatenate(xs, axis=-1)
        dmid = p['W1_rel'].shape[1]
        hdim = p['W2_rel'].shape[1]
        y1r, y1o = _tc_call(
            _pre_body,
            (jax.ShapeDtypeStruct((_N_PAD, dmid), jnp.float32),
             jax.ShapeDtypeStruct((_N_PAD, dmid), jnp.float32)),
            h_cat, p['W1_rel'], p['W1_root'], p['b1'].reshape(1, -1))
        acc1 = scatter_128(y1r, src1, dst1, zeros_128)
        y2r, y2o = _tc_call(
            _mid_body,
            (jax.ShapeDtypeStruct((_N_PAD, hdim), jnp.float32),
             jax.ShapeDtypeStruct((_N_PAD, hdim), jnp.float32)),
            acc1, y1o, p['g1'].reshape(1, -1), p['be1'].reshape(1, -1),
            p['W2_rel'], p['W2_root'], p['b2'].reshape(1, -1))
        acc2 = scatter_32(y2r, src2, dst2, zeros_32)
        h2 = _tc_call(
            _post_body,
            jax.ShapeDtypeStruct((_N_PAD, hdim), jnp.float32),
            acc2, y2o, p['g2'].reshape(1, -1), p['be2'].reshape(1, -1))
        xs.append(h2)

    return jnp.concatenate(xs, axis=-1)[:n]
